# initial kernel scaffold (unmeasured)
import jax
import jax.numpy as jnp
from jax import lax
from jax.experimental import pallas as pl
from jax.experimental.pallas import tpu as pltpu

N_DEV = 4


def kernel(x, w_mat):
    m_per, k = x.shape
    _, n_per = w_mat.shape

    def body(x_ref, w_ref, out_ref, comm, ring_ssem, ring_rsem,
             amax_buf, amax_ssem, amax_rsem):
        me = lax.axis_index("i")
        left = (me + N_DEV - 1) % N_DEV
        right = (me + 1) % N_DEV

        barrier = pltpu.get_barrier_semaphore()
        for nbr in (left, right):
            pl.semaphore_signal(
                barrier, inc=1,
                device_id=(nbr,), device_id_type=pl.DeviceIdType.MESH,
            )
        pl.semaphore_wait(barrier, 2)

        def gemm_relu(chunk):
            y = lax.dot_general(
                chunk, w_ref[...],
                (((1,), (0,)), ((), ())),
                precision=lax.Precision.HIGHEST,
                preferred_element_type=jnp.float32,
            )
            return jnp.maximum(y, 0.0)

        y0 = gemm_relu(x_ref[...])
        out_ref[pl.ds(me * m_per, m_per), :] = y0
        amax = jnp.max(y0)

        for h in range(N_DEV - 1):
            src = x_ref if h == 0 else comm.at[(h - 1) % 2]
            rdma = pltpu.make_async_remote_copy(
                src_ref=src,
                dst_ref=comm.at[h % 2],
                send_sem=ring_ssem.at[h],
                recv_sem=ring_rsem.at[h],
                device_id=(right,),
                device_id_type=pl.DeviceIdType.MESH,
            )
            rdma.start()
            rdma.wait()
            origin = (me + N_DEV - h - 1) % N_DEV
            yh = gemm_relu(comm[h % 2])
            out_ref[pl.ds(origin * m_per, m_per), :] = yh
            amax = jnp.maximum(amax, jnp.max(yh))

        amax_buf[me] = jnp.full((8, 128), amax, dtype=jnp.float32)
        sends = []
        for h in range(1, N_DEV):
            q = (me + h) % N_DEV
            send = pltpu.make_async_remote_copy(
                src_ref=amax_buf.at[me],
                dst_ref=amax_buf.at[me],
                send_sem=amax_ssem.at[h],
                recv_sem=amax_rsem.at[me],
                device_id=(q,),
                device_id_type=pl.DeviceIdType.MESH,
            )
            send.start()
            sends.append(send)
        for h in range(1, N_DEV):
            q = (me + h) % N_DEV
            recv = pltpu.make_async_remote_copy(
                src_ref=amax_buf.at[me],
                dst_ref=amax_buf.at[q],
                send_sem=amax_ssem.at[0],
                recv_sem=amax_rsem.at[q],
                device_id=(q,),
                device_id_type=pl.DeviceIdType.MESH,
            )
            recv.wait_recv()
        for send in sends:
            send.wait_send()

        amax_g = jnp.max(amax_buf[...])

        scale = amax_g / 448.0
        y = out_ref[...]
        q8 = jnp.clip(y / scale, 0.0, 448.0).astype(jnp.float8_e4m3fn)
        out_ref[...] = q8.astype(jnp.float32) * scale

    return pl.pallas_call(
        body,
        out_shape=jax.ShapeDtypeStruct((N_DEV * m_per, n_per), jnp.float32),
        in_specs=[
            pl.BlockSpec(memory_space=pltpu.VMEM),
            pl.BlockSpec(memory_space=pltpu.VMEM),
        ],
        out_specs=pl.BlockSpec(memory_space=pltpu.VMEM),
        scratch_shapes=[
            pltpu.VMEM((2, m_per, k), jnp.float32),
            pltpu.SemaphoreType.DMA((N_DEV - 1,)),
            pltpu.SemaphoreType.DMA((N_DEV - 1,)),
            pltpu.VMEM((N_DEV, 8, 128), jnp.float32),
            pltpu.SemaphoreType.DMA((N_DEV,)),
            pltpu.SemaphoreType.DMA((N_DEV,)),
        ],
        compiler_params=pltpu.CompilerParams(collective_id=0),
    )(x, w_mat)


# baseline (device time: 692349 ns/iter reference)
import jax
import jax.numpy as jnp
from jax import lax
from jax.experimental import pallas as pl
from jax.experimental.pallas import tpu as pltpu

N_DEV = 4


def kernel(x, w_mat):
    m_per, k = x.shape
    _, n_per = w_mat.shape

    def body(x_ref, w_ref, out_ref, comm, ring_ssem, ring_rsem,
             amax_buf, amax_ssem, amax_rsem):
        me = lax.axis_index("i")
        left = (me + N_DEV - 1) % N_DEV
        right = (me + 1) % N_DEV

        barrier = pltpu.get_barrier_semaphore()
        for nbr in (left, right):
            pl.semaphore_signal(
                barrier, inc=1,
                device_id=(nbr,), device_id_type=pl.DeviceIdType.MESH,
            )
        pl.semaphore_wait(barrier, 2)

        def gemm_relu(chunk):
            y = lax.dot_general(
                chunk, w_ref[...],
                (((1,), (0,)), ((), ())),
                precision=lax.Precision.HIGHEST,
                preferred_element_type=jnp.float32,
            )
            return jnp.maximum(y, 0.0)

        y0 = gemm_relu(x_ref[...])
        out_ref[pl.ds(me * m_per, m_per), :] = y0
        amax = jnp.max(y0)

        h_per = m_per // 2
        for h in range(2 * (N_DEV - 1)):
            if h < 2:
                src = x_ref.at[pl.ds(h * h_per, h_per)]
            else:
                src = comm.at[(h - 2) % 3]
            rdma = pltpu.make_async_remote_copy(
                src_ref=src,
                dst_ref=comm.at[h % 3],
                send_sem=ring_ssem.at[h],
                recv_sem=ring_rsem.at[h],
                device_id=(right,),
                device_id_type=pl.DeviceIdType.MESH,
            )
            rdma.start()
            rdma.wait()
            origin = (me + N_DEV - 1 - h // 2) % N_DEV
            yh = gemm_relu(comm[h % 3])
            out_ref[pl.ds(origin * m_per + (h % 2) * h_per, h_per), :] = yh
            amax = jnp.maximum(amax, jnp.max(yh))

        amax_buf[me] = jnp.full((8, 128), amax, dtype=jnp.float32)
        sends = []
        for h in range(1, N_DEV):
            q = (me + h) % N_DEV
            send = pltpu.make_async_remote_copy(
                src_ref=amax_buf.at[me],
                dst_ref=amax_buf.at[me],
                send_sem=amax_ssem.at[h],
                recv_sem=amax_rsem.at[me],
                device_id=(q,),
                device_id_type=pl.DeviceIdType.MESH,
            )
            send.start()
            sends.append(send)
        for h in range(1, N_DEV):
            q = (me + h) % N_DEV
            recv = pltpu.make_async_remote_copy(
                src_ref=amax_buf.at[me],
                dst_ref=amax_buf.at[q],
                send_sem=amax_ssem.at[0],
                recv_sem=amax_rsem.at[q],
                device_id=(q,),
                device_id_type=pl.DeviceIdType.MESH,
            )
            recv.wait_recv()
        for send in sends:
            send.wait_send()

        amax_g = jnp.max(amax_buf[...])

        scale = amax_g / 448.0
        y = out_ref[...]
        q8 = jnp.clip(y / scale, 0.0, 448.0).astype(jnp.float8_e4m3fn)
        out_ref[...] = q8.astype(jnp.float32) * scale

    return pl.pallas_call(
        body,
        out_shape=jax.ShapeDtypeStruct((N_DEV * m_per, n_per), jnp.float32),
        in_specs=[
            pl.BlockSpec(memory_space=pltpu.VMEM),
            pl.BlockSpec(memory_space=pltpu.VMEM),
        ],
        out_specs=pl.BlockSpec(memory_space=pltpu.VMEM),
        scratch_shapes=[
            pltpu.VMEM((3, m_per // 2, k), jnp.float32),
            pltpu.SemaphoreType.DMA((2 * (N_DEV - 1),)),
            pltpu.SemaphoreType.DMA((2 * (N_DEV - 1),)),
            pltpu.VMEM((N_DEV, 8, 128), jnp.float32),
            pltpu.SemaphoreType.DMA((N_DEV,)),
            pltpu.SemaphoreType.DMA((N_DEV,)),
        ],
        compiler_params=pltpu.CompilerParams(
            collective_id=0,
            vmem_limit_bytes=100 * 1024 * 1024,
        ),
    )(x, w_mat)


# device time: 235133 ns/iter; 2.9445x vs baseline; 2.9445x over previous
import jax
import jax.numpy as jnp
from jax import lax
from jax.experimental import pallas as pl
from jax.experimental.pallas import tpu as pltpu

N_DEV = 4

R0, R1, L0, L1 = 0, 1, 2, 3
T_FROM_LEFT, T_FROM_RIGHT, T_FAR_LO, T_FAR_HI = 0, 1, 2, 3


def kernel(x, w_mat):
    m_per, k = x.shape
    _, n_per = w_mat.shape
    n_half = n_per // 2

    def body(x_ref, w_ref, out_ref, wR, wL, wFar, tile_buf,
             wssem, wrsem, tssem, trsem, amax_buf, amax_ssem, amax_rsem):
        me = lax.axis_index("i")
        left = (me + N_DEV - 1) % N_DEV
        right = (me + 1) % N_DEV
        far = (me + 2) % N_DEV

        barrier = pltpu.get_barrier_semaphore()
        for nbr in (left, right):
            pl.semaphore_signal(
                barrier, inc=1,
                device_id=(nbr,), device_id_type=pl.DeviceIdType.MESH,
            )
        pl.semaphore_wait(barrier, 2)

        def copy(src, dst, ssem, rsem, dev):
            return pltpu.make_async_remote_copy(
                src_ref=src, dst_ref=dst, send_sem=ssem, recv_sem=rsem,
                device_id=(dev,), device_id_type=pl.DeviceIdType.MESH,
            )

        r0 = copy(w_ref, wR, wssem.at[R0], wrsem.at[R0], right)
        r0.start()
        l0 = copy(w_ref, wL, wssem.at[L0], wrsem.at[L0], left)
        l0.start()

        def gemm_relu(w_block):
            y = lax.dot_general(
                x_ref[...], w_block,
                (((1,), (0,)), ((), ())),
                precision=lax.Precision.HIGHEST,
                preferred_element_type=jnp.float32,
            )
            return jnp.maximum(y, 0.0)

        y_own = gemm_relu(w_ref[...])
        out_ref[pl.ds(me * m_per, m_per), :] = y_own
        amax = jnp.max(y_own)

        r0.wait_recv()
        r1 = copy(wR.at[:, pl.ds(0, n_half)], wFar.at[:, pl.ds(0, n_half)],
                  wssem.at[R1], wrsem.at[R1], right)
        r1.start()
        l0.wait_recv()
        l1 = copy(wL.at[:, pl.ds(n_half, n_half)],
                  wFar.at[:, pl.ds(n_half, n_half)],
                  wssem.at[L1], wrsem.at[L1], left)
        l1.start()

        tile_buf[0] = gemm_relu(wR[...])
        amax = jnp.maximum(amax, jnp.max(tile_buf[0]))
        t_l = copy(tile_buf.at[0], out_ref.at[pl.ds(me * m_per, m_per), :],
                   tssem.at[0], trsem.at[T_FROM_RIGHT], left)
        t_l.start()

        tile_buf[1] = gemm_relu(wL[...])
        amax = jnp.maximum(amax, jnp.max(tile_buf[1]))
        t_r = copy(tile_buf.at[1], out_ref.at[pl.ds(me * m_per, m_per), :],
                   tssem.at[1], trsem.at[T_FROM_LEFT], right)
        t_r.start()

        r1.wait_recv()
        l1.wait_recv()
        t_l.wait_send()
        tile_buf[0] = gemm_relu(wFar[...])
        amax = jnp.maximum(amax, jnp.max(tile_buf[0]))
        t_f0 = copy(tile_buf.at[0, :, pl.ds(0, n_half)],
                    out_ref.at[pl.ds(me * m_per, m_per), pl.ds(0, n_half)],
                    tssem.at[2], trsem.at[T_FAR_LO], far)
        t_f0.start()
        t_f1 = copy(tile_buf.at[0, :, pl.ds(n_half, n_half)],
                    out_ref.at[pl.ds(me * m_per, m_per), pl.ds(n_half, n_half)],
                    tssem.at[3], trsem.at[T_FAR_HI], far)
        t_f1.start()

        amax_buf[me] = jnp.full((8, 128), amax, dtype=jnp.float32)
        amax_sends = []
        for h in range(1, N_DEV):
            q = (me + h) % N_DEV
            send = copy(amax_buf.at[me], amax_buf.at[me],
                        amax_ssem.at[h - 1], amax_rsem.at[me], q)
            send.start()
            amax_sends.append(send)

        copy(tile_buf.at[0], out_ref.at[pl.ds(left * m_per, m_per), :],
             tssem.at[0], trsem.at[T_FROM_LEFT], left).wait_recv()
        copy(tile_buf.at[0], out_ref.at[pl.ds(right * m_per, m_per), :],
             tssem.at[0], trsem.at[T_FROM_RIGHT], right).wait_recv()
        copy(tile_buf.at[0, :, pl.ds(0, n_half)],
             out_ref.at[pl.ds(far * m_per, m_per), pl.ds(0, n_half)],
             tssem.at[0], trsem.at[T_FAR_LO], far).wait_recv()
        copy(tile_buf.at[0, :, pl.ds(n_half, n_half)],
             out_ref.at[pl.ds(far * m_per, m_per), pl.ds(n_half, n_half)],
             tssem.at[0], trsem.at[T_FAR_HI], far).wait_recv()
        for h in range(1, N_DEV):
            q = (me + h) % N_DEV
            copy(amax_buf.at[me], amax_buf.at[q],
                 amax_ssem.at[0], amax_rsem.at[q], q).wait_recv()

        amax_g = jnp.max(amax_buf[...])
        scale = amax_g / 448.0
        for b in range(N_DEV):
            y = out_ref[pl.ds(b * m_per, m_per), :]
            q8 = jnp.clip(y / scale, 0.0, 448.0).astype(jnp.float8_e4m3fn)
            out_ref[pl.ds(b * m_per, m_per), :] = q8.astype(jnp.float32) * scale

        for op in (r0, r1, l0, l1, t_r, t_f0, t_f1, *amax_sends):
            op.wait_send()

    return pl.pallas_call(
        body,
        out_shape=jax.ShapeDtypeStruct((N_DEV * m_per, n_per), jnp.float32),
        in_specs=[
            pl.BlockSpec(memory_space=pltpu.VMEM),
            pl.BlockSpec(memory_space=pltpu.VMEM),
        ],
        out_specs=pl.BlockSpec(memory_space=pltpu.VMEM),
        scratch_shapes=[
            pltpu.VMEM((k, n_per), jnp.float32),
            pltpu.VMEM((k, n_per), jnp.float32),
            pltpu.VMEM((k, n_per), jnp.float32),
            pltpu.VMEM((2, m_per, n_per), jnp.float32),
            pltpu.SemaphoreType.DMA((4,)),
            pltpu.SemaphoreType.DMA((4,)),
            pltpu.SemaphoreType.DMA((4,)),
            pltpu.SemaphoreType.DMA((4,)),
            pltpu.VMEM((N_DEV, 8, 128), jnp.float32),
            pltpu.SemaphoreType.DMA((3,)),
            pltpu.SemaphoreType.DMA((4,)),
        ],
        compiler_params=pltpu.CompilerParams(
            collective_id=0,
            vmem_limit_bytes=100 * 1024 * 1024,
        ),
    )(x, w_mat)


# device time: 206914 ns/iter; 3.3461x vs baseline; 1.1364x over previous
import jax
import jax.numpy as jnp
from jax import lax
from jax.experimental import pallas as pl
from jax.experimental.pallas import tpu as pltpu

N_DEV = 4

R0, R1, L0, L1 = 0, 1, 2, 3
T_FROM_LEFT, T_FROM_RIGHT, T_FAR_LO, T_FAR_HI = 0, 1, 2, 3


def kernel(x, w_mat):
    m_per, k = x.shape
    _, n_per = w_mat.shape
    n_half = n_per // 2

    def body(x_ref, w_ref, out_ref, wR, wL, wFar, tile_buf,
             wssem, wrsem, tssem, trsem, amax_buf, amax_ssem, amax_rsem):
        me = lax.axis_index("i")
        left = (me + N_DEV - 1) % N_DEV
        right = (me + 1) % N_DEV
        far = (me + 2) % N_DEV

        barrier = pltpu.get_barrier_semaphore()
        for nbr in (left, right):
            pl.semaphore_signal(
                barrier, inc=1,
                device_id=(nbr,), device_id_type=pl.DeviceIdType.MESH,
            )
        pl.semaphore_wait(barrier, 2)

        def copy(src, dst, ssem, rsem, dev):
            return pltpu.make_async_remote_copy(
                src_ref=src, dst_ref=dst, send_sem=ssem, recv_sem=rsem,
                device_id=(dev,), device_id_type=pl.DeviceIdType.MESH,
            )

        r0 = copy(w_ref, wR, wssem.at[R0], wrsem.at[R0], right)
        r0.start()
        l0 = copy(w_ref, wL, wssem.at[L0], wrsem.at[L0], left)
        l0.start()

        def gemm_relu(w_block):
            y = lax.dot_general(
                x_ref[...], w_block,
                (((1,), (0,)), ((), ())),
                precision=lax.Precision.DEFAULT,
                preferred_element_type=jnp.float32,
            )
            return jnp.maximum(y, 0.0)

        y_own = gemm_relu(w_ref[...])
        out_ref[pl.ds(me * m_per, m_per), :] = y_own
        amax = jnp.max(y_own)

        r0.wait_recv()
        r1 = copy(wR.at[:, pl.ds(0, n_half)], wFar.at[:, pl.ds(0, n_half)],
                  wssem.at[R1], wrsem.at[R1], right)
        r1.start()
        l0.wait_recv()
        l1 = copy(wL.at[:, pl.ds(n_half, n_half)],
                  wFar.at[:, pl.ds(n_half, n_half)],
                  wssem.at[L1], wrsem.at[L1], left)
        l1.start()

        tile_buf[0] = gemm_relu(wR[...])
        amax = jnp.maximum(amax, jnp.max(tile_buf[0]))
        t_l = copy(tile_buf.at[0], out_ref.at[pl.ds(me * m_per, m_per), :],
                   tssem.at[0], trsem.at[T_FROM_RIGHT], left)
        t_l.start()

        tile_buf[1] = gemm_relu(wL[...])
        amax = jnp.maximum(amax, jnp.max(tile_buf[1]))
        t_r = copy(tile_buf.at[1], out_ref.at[pl.ds(me * m_per, m_per), :],
                   tssem.at[1], trsem.at[T_FROM_LEFT], right)
        t_r.start()

        r1.wait_recv()
        l1.wait_recv()
        r1.wait_send()
        wR[pl.ds(0, m_per), :] = gemm_relu(wFar[...])
        amax = jnp.maximum(amax, jnp.max(wR[pl.ds(0, m_per), :]))
        t_f0 = copy(wR.at[pl.ds(0, m_per), pl.ds(0, n_half)],
                    out_ref.at[pl.ds(me * m_per, m_per), pl.ds(0, n_half)],
                    tssem.at[2], trsem.at[T_FAR_LO], far)
        t_f0.start()
        t_f1 = copy(wR.at[pl.ds(0, m_per), pl.ds(n_half, n_half)],
                    out_ref.at[pl.ds(me * m_per, m_per), pl.ds(n_half, n_half)],
                    tssem.at[3], trsem.at[T_FAR_HI], far)
        t_f1.start()

        amax_buf[me] = jnp.full((8, 128), amax, dtype=jnp.float32)
        amax_sends = []
        for h in range(1, N_DEV):
            q = (me + h) % N_DEV
            send = copy(amax_buf.at[me], amax_buf.at[me],
                        amax_ssem.at[h - 1], amax_rsem.at[me], q)
            send.start()
            amax_sends.append(send)

        copy(tile_buf.at[0], out_ref.at[pl.ds(left * m_per, m_per), :],
             tssem.at[0], trsem.at[T_FROM_LEFT], left).wait_recv()
        copy(tile_buf.at[0], out_ref.at[pl.ds(right * m_per, m_per), :],
             tssem.at[0], trsem.at[T_FROM_RIGHT], right).wait_recv()
        copy(tile_buf.at[0, :, pl.ds(0, n_half)],
             out_ref.at[pl.ds(far * m_per, m_per), pl.ds(0, n_half)],
             tssem.at[0], trsem.at[T_FAR_LO], far).wait_recv()
        copy(tile_buf.at[0, :, pl.ds(n_half, n_half)],
             out_ref.at[pl.ds(far * m_per, m_per), pl.ds(n_half, n_half)],
             tssem.at[0], trsem.at[T_FAR_HI], far).wait_recv()
        for h in range(1, N_DEV):
            q = (me + h) % N_DEV
            copy(amax_buf.at[me], amax_buf.at[q],
                 amax_ssem.at[0], amax_rsem.at[q], q).wait_recv()

        amax_g = jnp.max(amax_buf[...])
        scale = amax_g / 448.0
        for b in range(N_DEV):
            y = out_ref[pl.ds(b * m_per, m_per), :]
            q8 = jnp.clip(y / scale, 0.0, 448.0).astype(jnp.float8_e4m3fn)
            out_ref[pl.ds(b * m_per, m_per), :] = q8.astype(jnp.float32) * scale

        for op in (r0, l0, l1, t_l, t_r, t_f0, t_f1, *amax_sends):
            op.wait_send()

    return pl.pallas_call(
        body,
        out_shape=jax.ShapeDtypeStruct((N_DEV * m_per, n_per), jnp.float32),
        in_specs=[
            pl.BlockSpec(memory_space=pltpu.VMEM),
            pl.BlockSpec(memory_space=pltpu.VMEM),
        ],
        out_specs=pl.BlockSpec(memory_space=pltpu.VMEM),
        scratch_shapes=[
            pltpu.VMEM((k, n_per), jnp.float32),
            pltpu.VMEM((k, n_per), jnp.float32),
            pltpu.VMEM((k, n_per), jnp.float32),
            pltpu.VMEM((2, m_per, n_per), jnp.float32),
            pltpu.SemaphoreType.DMA((4,)),
            pltpu.SemaphoreType.DMA((4,)),
            pltpu.SemaphoreType.DMA((4,)),
            pltpu.SemaphoreType.DMA((4,)),
            pltpu.VMEM((N_DEV, 8, 128), jnp.float32),
            pltpu.SemaphoreType.DMA((3,)),
            pltpu.SemaphoreType.DMA((4,)),
        ],
        compiler_params=pltpu.CompilerParams(
            collective_id=0,
            vmem_limit_bytes=100 * 1024 * 1024,
        ),
    )(x, w_mat)


# device time: 139650 ns/iter; 4.9577x vs baseline; 1.4817x over previous
import jax
import jax.numpy as jnp
from jax import lax
from jax.experimental import pallas as pl
from jax.experimental.pallas import tpu as pltpu

N_DEV = 4

R0, R1, L0, L1 = 0, 1, 2, 3
T_FROM_LEFT, T_FROM_RIGHT, T_FAR_LO, T_FAR_HI = 0, 1, 2, 3


def kernel(x, w_mat):
    m_per, k = x.shape
    _, n_per = w_mat.shape
    n_half = n_per // 2

    def body(x_ref, w_ref, out_ref, xb, wb, wR, wL, wFar,
             wssem, wrsem, tssem, trsem, amax_buf, amax_ssem, amax_rsem):
        me = lax.axis_index("i")
        left = (me + N_DEV - 1) % N_DEV
        right = (me + 1) % N_DEV
        far = (me + 2) % N_DEV

        wb[...] = w_ref[...].astype(jnp.bfloat16)

        barrier = pltpu.get_barrier_semaphore()
        for nbr in (left, right):
            pl.semaphore_signal(
                barrier, inc=1,
                device_id=(nbr,), device_id_type=pl.DeviceIdType.MESH,
            )
        pl.semaphore_wait(barrier, 2)

        def copy(src, dst, ssem, rsem, dev):
            return pltpu.make_async_remote_copy(
                src_ref=src, dst_ref=dst, send_sem=ssem, recv_sem=rsem,
                device_id=(dev,), device_id_type=pl.DeviceIdType.MESH,
            )

        r0 = copy(wb, wR, wssem.at[R0], wrsem.at[R0], right)
        r0.start()
        l0 = copy(wb, wL, wssem.at[L0], wrsem.at[L0], left)
        l0.start()

        xb[...] = x_ref[...].astype(jnp.bfloat16)

        def gemm_relu(w_block):
            y = lax.dot_general(
                xb[...], w_block,
                (((1,), (0,)), ((), ())),
                preferred_element_type=jnp.float32,
            )
            return jnp.maximum(y, 0.0)

        y_own = gemm_relu(wb[...])
        out_ref[pl.ds(me * m_per, m_per), :] = y_own
        amax = jnp.max(y_own)

        r0.wait_recv()
        r1 = copy(wR.at[:, pl.ds(0, n_half)], wFar.at[:, pl.ds(0, n_half)],
                  wssem.at[R1], wrsem.at[R1], right)
        r1.start()
        l0.wait_recv()
        l1 = copy(wL.at[:, pl.ds(n_half, n_half)],
                  wFar.at[:, pl.ds(n_half, n_half)],
                  wssem.at[L1], wrsem.at[L1], left)
        l1.start()

        x_ref[:, pl.ds(0, n_per)] = gemm_relu(wR[...])
        amax = jnp.maximum(amax, jnp.max(x_ref[:, pl.ds(0, n_per)]))
        t_l = copy(x_ref.at[:, pl.ds(0, n_per)],
                   out_ref.at[pl.ds(me * m_per, m_per), :],
                   tssem.at[0], trsem.at[T_FROM_RIGHT], left)
        t_l.start()

        x_ref[:, pl.ds(n_per, n_per)] = gemm_relu(wL[...])
        amax = jnp.maximum(amax, jnp.max(x_ref[:, pl.ds(n_per, n_per)]))
        t_r = copy(x_ref.at[:, pl.ds(n_per, n_per)],
                   out_ref.at[pl.ds(me * m_per, m_per), :],
                   tssem.at[1], trsem.at[T_FROM_LEFT], right)
        t_r.start()

        r1.wait_recv()
        l1.wait_recv()
        x_ref[:, pl.ds(2 * n_per, n_per)] = gemm_relu(wFar[...])
        amax = jnp.maximum(amax, jnp.max(x_ref[:, pl.ds(2 * n_per, n_per)]))
        t_f0 = copy(x_ref.at[:, pl.ds(2 * n_per, n_half)],
                    out_ref.at[pl.ds(me * m_per, m_per), pl.ds(0, n_half)],
                    tssem.at[2], trsem.at[T_FAR_LO], far)
        t_f0.start()
        t_f1 = copy(x_ref.at[:, pl.ds(2 * n_per + n_half, n_half)],
                    out_ref.at[pl.ds(me * m_per, m_per), pl.ds(n_half, n_half)],
                    tssem.at[3], trsem.at[T_FAR_HI], far)
        t_f1.start()

        amax_buf[me] = jnp.full((8, 128), amax, dtype=jnp.float32)
        amax_sends = []
        for h in range(1, N_DEV):
            q = (me + h) % N_DEV
            send = copy(amax_buf.at[me], amax_buf.at[me],
                        amax_ssem.at[h - 1], amax_rsem.at[me], q)
            send.start()
            amax_sends.append(send)
        for h in range(1, N_DEV):
            q = (me + h) % N_DEV
            copy(amax_buf.at[me], amax_buf.at[q],
                 amax_ssem.at[0], amax_rsem.at[q], q).wait_recv()
        amax_g = jnp.max(amax_buf[...])
        scale = amax_g / 448.0

        def quantize_block(row0):
            y = out_ref[pl.ds(row0, m_per), :]
            q8 = jnp.clip(y / scale, 0.0, 448.0).astype(jnp.float8_e4m3fn)
            out_ref[pl.ds(row0, m_per), :] = q8.astype(jnp.float32) * scale

        quantize_block(me * m_per)
        copy(x_ref.at[:, pl.ds(0, n_per)],
             out_ref.at[pl.ds(left * m_per, m_per), :],
             tssem.at[0], trsem.at[T_FROM_LEFT], left).wait_recv()
        quantize_block(left * m_per)
        copy(x_ref.at[:, pl.ds(0, n_per)],
             out_ref.at[pl.ds(right * m_per, m_per), :],
             tssem.at[0], trsem.at[T_FROM_RIGHT], right).wait_recv()
        quantize_block(right * m_per)
        copy(x_ref.at[:, pl.ds(0, n_half)],
             out_ref.at[pl.ds(far * m_per, m_per), pl.ds(0, n_half)],
             tssem.at[0], trsem.at[T_FAR_LO], far).wait_recv()
        copy(x_ref.at[:, pl.ds(0, n_half)],
             out_ref.at[pl.ds(far * m_per, m_per), pl.ds(n_half, n_half)],
             tssem.at[0], trsem.at[T_FAR_HI], far).wait_recv()
        quantize_block(far * m_per)

        for op in (r0, r1, l0, l1, t_l, t_r, t_f0, t_f1, *amax_sends):
            op.wait_send()

    return pl.pallas_call(
        body,
        out_shape=jax.ShapeDtypeStruct((N_DEV * m_per, n_per), jnp.float32),
        in_specs=[
            pl.BlockSpec(memory_space=pltpu.VMEM),
            pl.BlockSpec(memory_space=pltpu.VMEM),
        ],
        out_specs=pl.BlockSpec(memory_space=pltpu.VMEM),
        scratch_shapes=[
            pltpu.VMEM((m_per, k), jnp.bfloat16),
            pltpu.VMEM((k, n_per), jnp.bfloat16),
            pltpu.VMEM((k, n_per), jnp.bfloat16),
            pltpu.VMEM((k, n_per), jnp.bfloat16),
            pltpu.VMEM((k, n_per), jnp.bfloat16),
            pltpu.SemaphoreType.DMA((4,)),
            pltpu.SemaphoreType.DMA((4,)),
            pltpu.SemaphoreType.DMA((4,)),
            pltpu.SemaphoreType.DMA((4,)),
            pltpu.VMEM((N_DEV, 8, 128), jnp.float32),
            pltpu.SemaphoreType.DMA((3,)),
            pltpu.SemaphoreType.DMA((4,)),
        ],
        compiler_params=pltpu.CompilerParams(
            collective_id=0,
            vmem_limit_bytes=100 * 1024 * 1024,
        ),
    )(x, w_mat)
